# baseline (speedup 1.0000x reference)
import functools

import jax
import jax.numpy as jnp
from jax import lax
from jax.experimental import pallas as pl
from jax.experimental.pallas import tpu as pltpu
from jax.experimental.pallas import tpu_sc as plsc

N_NODES = 10000
N_HE = 5000
NNZ = 320000
D = 128
TRAIN = 5000

NC = 2
NS = 16
NW = NC * NS
PW = NNZ // NS
EHALF = N_HE // 2
EACC = 2560
EGAP = EACC - EHALF
VHALF = N_NODES // 2
VACC = 5120
MOLPAD = 10240

_mesh = plsc.VectorSubcoreMesh(core_axis_name="c", subcore_axis_name="s")


def _zero_rows(ref, nrows, ncols):
    z = jnp.zeros((16,), jnp.float32)

    def row(r, _):
        for cc in range(ncols // 16):
            ref[r, pl.ds(cc * 16, 16)] = z
        return 0

    lax.fori_loop(0, nrows, row, 0)


def _make_pass(acc_rows, half, src_shift, with_counts, with_mol, k):
    rps = acc_rows // NS
    trash = acc_rows - 1
    iters = PW // k
    tail = PW - iters * k
    sch = 160
    nst = rps // sch

    out_ty = [jax.ShapeDtypeStruct((NC, acc_rows, D), jnp.float32)]
    if with_counts:
        out_ty.append(jax.ShapeDtypeStruct((NC, acc_rows, D), jnp.float32))
    if with_mol:
        out_ty.append(jax.ShapeDtypeStruct((MOLPAD, D), jnp.float32))
    scratch = [
        pltpu.VMEM((k,), jnp.int32),
        pltpu.VMEM((k,), jnp.int32),
        pltpu.VMEM((k, D), jnp.float32),
        pltpu.VMEM((sch, D), jnp.float32),
        pltpu.VMEM_SHARED((acc_rows, D), jnp.float32),
        pltpu.SemaphoreType.DMA,
    ]
    if with_mol:
        scratch += [pltpu.VMEM((sch,), jnp.int32)]

    @functools.partial(pl.kernel, mesh=_mesh, out_type=out_ty,
                       scratch_types=scratch)
    def _p(*args):
        it = iter(args)
        tab, sidx, didx = next(it), next(it), next(it)
        if with_mol:
            drug, molidx = next(it), next(it)
        acc_o = next(it)
        cnt_o = next(it) if with_counts else None
        mol_o = next(it) if with_mol else None
        sb0, db0, rows0, stage, acc, sem0 = (
            next(it), next(it), next(it), next(it), next(it), next(it))
        if with_mol:
            molb = next(it)
        sbs, dbs = (sb0, sb0), (db0, db0)

        c = lax.axis_index("c")
        s = lax.axis_index("s")
        lo = c * half
        tr = jnp.full((16,), trash, jnp.int32)

        def zero_acc():
            _zero_rows(stage, sch, D)
            for j in range(nst):
                pltpu.sync_copy(stage, acc.at[pl.ds(s * rps + j * sch, sch)])

        def load_idx(i, b):
            base = s * PW + i * k
            pltpu.sync_copy(sidx.at[pl.ds(base, k)], sbs[b])
            pltpu.sync_copy(didx.at[pl.ds(base, k)], dbs[b])

        far = jnp.full((16,), 1 << 20, jnp.int32)
        zi = jnp.zeros((16,), jnp.int32)

        def load_tail(b, dest_only=False):
            base = s * PW + iters * k
            if not dest_only:
                pltpu.sync_copy(sidx.at[pl.ds(base, tail)],
                                sbs[b].at[pl.ds(0, tail)])
            pltpu.sync_copy(didx.at[pl.ds(base, tail)],
                            dbs[b].at[pl.ds(0, tail)])
            for g in range(tail // 16, k // 16):
                if not dest_only:
                    sbs[b][pl.ds(g * 16, 16)] = zi
                dbs[b][pl.ds(g * 16, 16)] = far

        def remap(b, dest_only=False):
            dbr, sbr = dbs[b], sbs[b]

            def grp(g, __):
                d = dbr[pl.ds(g * 16, 16)]
                dl = d - lo
                ok = (dl >= 0) & (dl < half)
                dbr[pl.ds(g * 16, 16)] = jnp.where(ok, dl, tr)
                if src_shift and not dest_only:
                    x = sbr[pl.ds(g * 16, 16)]
                    sbr[pl.ds(g * 16, 16)] = jnp.where(
                        x >= src_shift[0], x + src_shift[1], x)
                return 0

            lax.fori_loop(0, k // 16, grp, 0)

        def stage_out(dst_o):
            for j in range(nst):
                pltpu.sync_copy(acc.at[pl.ds(s * rps + j * sch, sch)], stage)
                pltpu.sync_copy(stage,
                                dst_o.at[c, pl.ds(s * rps + j * sch, sch)])

        zero_acc()
        plsc.subcore_barrier()

        def body(i, _):
            load_idx(i, 0)
            remap(0)
            pltpu.async_copy(tab.at[sb0], rows0, sem0).wait()
            pltpu.sync_copy(rows0, acc.at[db0], add=True)
            return 0

        lax.fori_loop(0, iters, body, 0)

        if tail:
            load_tail(0)
            remap(0)
            pltpu.async_copy(tab.at[sb0], rows0, sem0).wait()
            pltpu.sync_copy(rows0, acc.at[db0], add=True)

        if with_mol:
            wid = s * NC + c
            mw = MOLPAD // NW
            for j in range(mw // sch):
                off = wid * mw + j * sch
                pltpu.sync_copy(molidx.at[pl.ds(off, sch)], molb)
                pltpu.async_copy(drug.at[molb], stage, sem0).wait()
                pltpu.sync_copy(stage, mol_o.at[pl.ds(off, sch)])

        plsc.subcore_barrier()
        stage_out(acc_o)

        if with_counts:
            plsc.subcore_barrier()
            zero_acc()
            one = jnp.ones((16,), jnp.float32)

            def orow(r, _):
                for cc in range(D // 16):
                    rows0[r, pl.ds(cc * 16, 16)] = one
                return 0

            lax.fori_loop(0, k, orow, 0)
            plsc.subcore_barrier()

            def body2(i, _):
                base = s * PW + i * k
                pltpu.sync_copy(didx.at[pl.ds(base, k)], db0)
                remap(0, dest_only=True)
                pltpu.sync_copy(rows0, acc.at[db0], add=True)
                return 0

            lax.fori_loop(0, iters, body2, 0)
            if tail:
                load_tail(0, dest_only=True)
                remap(0, dest_only=True)
                pltpu.sync_copy(rows0, acc.at[db0], add=True)
            plsc.subcore_barrier()
            stage_out(cnt_o)

    return _p


_p1 = _make_pass(EACC, EHALF, None, True, True, 480)
_p2 = _make_pass(VACC, VHALF, (EHALF, EGAP), True, False, 480)
_p3 = _make_pass(EACC, EHALF, None, False, False, 480)
_p4 = _make_pass(VACC, VHALF, (EHALF, EGAP), False, False, 480)


def _mm_relu_body(x_ref, w_ref, b_ref, o_ref):
    o_ref[...] = jax.nn.relu(
        jnp.dot(x_ref[...], w_ref[...], preferred_element_type=jnp.float32)
        + b_ref[...])


def _relu_mm(x, w, b):
    n = x.shape[0]
    bs = 1000
    return pl.pallas_call(
        _mm_relu_body,
        grid=(n // bs,),
        in_specs=[pl.BlockSpec((bs, D), lambda i: (i, 0)),
                  pl.BlockSpec((D, D), lambda i: (0, 0)),
                  pl.BlockSpec((1, D), lambda i: (0, 0))],
        out_specs=pl.BlockSpec((bs, D), lambda i: (i, 0)),
        out_shape=jax.ShapeDtypeStruct((n, D), jnp.float32),
    )(x, w, b.reshape(1, D))


def _cnt_col(c_ref):
    return jnp.maximum(c_ref[0, :, :1], 1.0)


def _div_e_body(s_ref, c_ref, o_ref):
    o_ref[...] = s_ref[0] / _cnt_col(c_ref)


def _div_e(esum, ecnt):
    return pl.pallas_call(
        _div_e_body,
        grid=(NC,),
        in_specs=[pl.BlockSpec((1, EACC, D), lambda i: (i, 0, 0)),
                  pl.BlockSpec((1, EACC, D), lambda i: (i, 0, 0))],
        out_specs=pl.BlockSpec((EACC, D), lambda i: (i, 0)),
        out_shape=jax.ShapeDtypeStruct((NC * EACC, D), jnp.float32),
    )(esum, ecnt)


def _xw_body(s_ref, c_ref, w_ref, b_ref, o_ref):
    x = jax.nn.relu(s_ref[0] / _cnt_col(c_ref))[:VHALF]
    o_ref[...] = jax.nn.relu(
        jnp.dot(x, w_ref[...], preferred_element_type=jnp.float32)
        + b_ref[...])


def _xw(vsum, vcnt, w, b):
    return pl.pallas_call(
        _xw_body,
        grid=(NC,),
        in_specs=[pl.BlockSpec((1, VACC, D), lambda i: (i, 0, 0)),
                  pl.BlockSpec((1, VACC, D), lambda i: (i, 0, 0)),
                  pl.BlockSpec((D, D), lambda i: (0, 0)),
                  pl.BlockSpec((1, D), lambda i: (0, 0))],
        out_specs=pl.BlockSpec((VHALF, D), lambda i: (i, 0)),
        out_shape=jax.ShapeDtypeStruct((N_NODES, D), jnp.float32),
    )(vsum, vcnt, w, b.reshape(1, D))


def _feat_body(s_ref, c_ref, m_ref, o_ref):
    o_ref[...] = (s_ref[0] / _cnt_col(c_ref))[:VHALF] + m_ref[...]


def _feat_comb(vsum, vcnt, mol):
    return pl.pallas_call(
        _feat_body,
        grid=(NC,),
        in_specs=[pl.BlockSpec((1, VACC, D), lambda i: (i, 0, 0)),
                  pl.BlockSpec((1, VACC, D), lambda i: (i, 0, 0)),
                  pl.BlockSpec((VHALF, D), lambda i: (i, 0))],
        out_specs=pl.BlockSpec((VHALF, D), lambda i: (i, 0)),
        out_shape=jax.ShapeDtypeStruct((N_NODES, D), jnp.float32),
    )(vsum, vcnt, mol)


def _tail_body(f_ref, yb_ref, wmua, wmub, bmu, wlva, wlvb, blv,
               w3, b3, g2, beta, wc, bc, wda, wdb, bd,
               mu_ref, lv_ref, lg_ref, rc_ref):
    f = f_ref[...]
    yb = yb_ref[...]
    dot = functools.partial(jnp.dot, preferred_element_type=jnp.float32)
    mu = dot(f, wmua[...]) + dot(yb, wmub[...]) + bmu[...]
    lv = dot(f, wlva[...]) + dot(yb, wlvb[...]) + blv[...]
    h = dot(mu, w3[...]) + b3[...]
    h = jax.nn.relu(g2[...] * h + beta[...])
    lg = dot(h, wc[...]) + bc[...]
    rc = dot(mu, wda[...]) + dot(yb, wdb[...]) + bd[...]
    mu_ref[...] = mu
    lv_ref[...] = lv
    lg_ref[...] = lg
    rc_ref[...] = rc


def _tail(feat, y_bin, Wmu, bmu, Wlv, blv, W3, b3, gamma, beta, Wc, bc,
          Wd, bd):
    bs = 1000
    n = N_NODES - TRAIN
    g2 = (gamma / jnp.sqrt(1.0 + 1e-5)).reshape(1, 64)
    full = lambda *s: pl.BlockSpec(s, lambda i: tuple(0 for _ in s))
    return pl.pallas_call(
        _tail_body,
        grid=(n // bs,),
        in_specs=[
            pl.BlockSpec((bs, D), lambda i: (i + TRAIN // bs, 0)),
            pl.BlockSpec((bs, 3), lambda i: (i, 0)),
            full(D, 64), full(3, 64), full(1, 64),
            full(D, 64), full(3, 64), full(1, 64),
            full(64, 64), full(1, 64),
            full(1, 64), full(1, 64),
            full(64, 3), full(1, 3),
            full(64, D), full(3, D), full(1, D),
        ],
        out_specs=[pl.BlockSpec((bs, 64), lambda i: (i, 0)),
                   pl.BlockSpec((bs, 64), lambda i: (i, 0)),
                   pl.BlockSpec((bs, 3), lambda i: (i, 0)),
                   pl.BlockSpec((bs, D), lambda i: (i, 0))],
        out_shape=[jax.ShapeDtypeStruct((n, 64), jnp.float32),
                   jax.ShapeDtypeStruct((n, 64), jnp.float32),
                   jax.ShapeDtypeStruct((n, 3), jnp.float32),
                   jax.ShapeDtypeStruct((n, D), jnp.float32)],
    )(feat, y_bin,
      Wmu[:D], Wmu[D:], bmu.reshape(1, 64),
      Wlv[:D], Wlv[D:], blv.reshape(1, 64),
      W3, b3.reshape(1, 64), g2, beta.reshape(1, 64),
      Wc, bc.reshape(1, 3), Wd[:64], Wd[64:], bd.reshape(1, D))


def kernel(feature, v_idx, e_idx, y_bin, y_target, drug_matrix, new_data_idx,
           W1, b1, W2, b2, Wmu, bmu, Wlv, blv, W3, b3, gamma, beta, Wc, bc,
           Wd, bd):
    vi = v_idx.astype(jnp.int32)
    ei = e_idx.astype(jnp.int32)
    molidx = jnp.concatenate(
        [new_data_idx.astype(jnp.int32),
         jnp.zeros((MOLPAD - N_NODES,), jnp.int32)])

    A1 = _relu_mm(feature, W1, b1)
    es1, ec, mol = _p1(A1, vi, ei, drug_matrix, molidx)
    E1 = _div_e(es1, ec)
    vs1, vc = _p2(E1, ei, vi)
    A2 = _xw(vs1, vc, W2, b2)
    es2, = _p3(A2, vi, ei)
    E2 = _div_e(es2, ec)
    vs2, = _p4(E2, ei, vi)
    feat = _feat_comb(vs2, vc, mol[:N_NODES])
    mu, lv, lg, rc = _tail(feat, y_bin, Wmu, bmu, Wlv, blv, W3, b3,
                           gamma, beta, Wc, bc, Wd, bd)
    return (mu, lv, mu, lg, rc, y_target, feat)


# speedup vs baseline: 1.3902x; 1.3902x over previous
import functools

import jax
import jax.numpy as jnp
from jax import lax
from jax.experimental import pallas as pl
from jax.experimental.pallas import tpu as pltpu
from jax.experimental.pallas import tpu_sc as plsc

N_NODES = 10000
N_HE = 5000
NNZ = 320000
D = 128
TRAIN = 5000

NC = 2
NS = 16
NW = NC * NS
PW = NNZ // NS
EHALF = N_HE // 2
EACC = 2560
EGAP = EACC - EHALF
VHALF = N_NODES // 2
VACC = 5120
MOLPAD = 10240

_mesh = plsc.VectorSubcoreMesh(core_axis_name="c", subcore_axis_name="s")


def _zero_rows(ref, nrows, ncols):
    z = jnp.zeros((16,), jnp.float32)

    def row(r, _):
        for cc in range(ncols // 16):
            ref[r, pl.ds(cc * 16, 16)] = z
        return 0

    lax.fori_loop(0, nrows, row, 0)


def _make_pass(acc_rows, half, src_shift, with_counts, with_mol, k):
    rps = acc_rows // NS
    trash = acc_rows - 1
    iters = PW // k
    tail = PW - iters * k
    sch = 160
    nst = rps // sch

    out_ty = [jax.ShapeDtypeStruct((NC, acc_rows, D), jnp.float32)]
    if with_counts:
        out_ty.append(jax.ShapeDtypeStruct((NC, acc_rows, D), jnp.float32))
    if with_mol:
        out_ty.append(jax.ShapeDtypeStruct((MOLPAD, D), jnp.float32))
    scratch = [
        pltpu.VMEM((k,), jnp.int32),
        pltpu.VMEM((k,), jnp.int32),
        pltpu.VMEM((k, D), jnp.float32),
        pltpu.VMEM((sch, D), jnp.float32),
        pltpu.VMEM_SHARED((acc_rows, D), jnp.float32),
        pltpu.SemaphoreType.DMA,
    ]
    if with_mol:
        scratch += [pltpu.VMEM((sch,), jnp.int32)]

    @functools.partial(pl.kernel, mesh=_mesh, out_type=out_ty,
                       scratch_types=scratch)
    def _p(*args):
        it = iter(args)
        tab, sidx, didx = next(it), next(it), next(it)
        if with_mol:
            drug, molidx = next(it), next(it)
        acc_o = next(it)
        cnt_o = next(it) if with_counts else None
        mol_o = next(it) if with_mol else None
        sb0, db0, rows0, stage, acc, sem0 = (
            next(it), next(it), next(it), next(it), next(it), next(it))
        if with_mol:
            molb = next(it)
        sbs, dbs = (sb0, sb0), (db0, db0)

        c = lax.axis_index("c")
        s = lax.axis_index("s")
        lo = c * half
        tr = jnp.full((16,), trash, jnp.int32)

        def zero_acc():
            _zero_rows(stage, sch, D)
            for j in range(nst):
                pltpu.sync_copy(stage, acc.at[pl.ds(s * rps + j * sch, sch)])

        def load_idx(i, b):
            base = s * PW + i * k
            pltpu.sync_copy(sidx.at[pl.ds(base, k)], sbs[b])
            pltpu.sync_copy(didx.at[pl.ds(base, k)], dbs[b])

        far = jnp.full((16,), 1 << 20, jnp.int32)
        zi = jnp.zeros((16,), jnp.int32)

        def load_tail(b, dest_only=False):
            base = s * PW + iters * k
            if not dest_only:
                pltpu.sync_copy(sidx.at[pl.ds(base, tail)],
                                sbs[b].at[pl.ds(0, tail)])
            pltpu.sync_copy(didx.at[pl.ds(base, tail)],
                            dbs[b].at[pl.ds(0, tail)])
            for g in range(tail // 16, k // 16):
                if not dest_only:
                    sbs[b][pl.ds(g * 16, 16)] = zi
                dbs[b][pl.ds(g * 16, 16)] = far

        def remap(b, dest_only=False):
            dbr, sbr = dbs[b], sbs[b]

            def grp(g, __):
                d = dbr[pl.ds(g * 16, 16)]
                dl = d - lo
                ok = (dl >= 0) & (dl < half)
                dbr[pl.ds(g * 16, 16)] = jnp.where(ok, dl, tr)
                if src_shift and not dest_only:
                    x = sbr[pl.ds(g * 16, 16)]
                    sbr[pl.ds(g * 16, 16)] = jnp.where(
                        x >= src_shift[0], x + src_shift[1], x)
                return 0

            lax.fori_loop(0, k // 16, grp, 0)

        def stage_out(dst_o):
            for j in range(nst):
                pltpu.sync_copy(acc.at[pl.ds(s * rps + j * sch, sch)], stage)
                pltpu.sync_copy(stage,
                                dst_o.at[c, pl.ds(s * rps + j * sch, sch)])

        zero_acc()
        plsc.subcore_barrier()

        def body(i, _):
            load_idx(i, 0)
            remap(0)
            pltpu.async_copy(tab.at[sb0], rows0, sem0).wait()
            pltpu.sync_copy(rows0, acc.at[db0], add=True)
            return 0

        lax.fori_loop(0, iters, body, 0)

        if tail:
            load_tail(0)
            remap(0)
            pltpu.async_copy(tab.at[sb0], rows0, sem0).wait()
            pltpu.sync_copy(rows0, acc.at[db0], add=True)

        if with_mol:
            wid = s * NC + c
            mw = MOLPAD // NW
            for j in range(mw // sch):
                off = wid * mw + j * sch
                pltpu.sync_copy(molidx.at[pl.ds(off, sch)], molb)
                pltpu.async_copy(drug.at[molb], stage, sem0).wait()
                pltpu.sync_copy(stage, mol_o.at[pl.ds(off, sch)])

        plsc.subcore_barrier()
        stage_out(acc_o)

        if with_counts:
            plsc.subcore_barrier()
            zero_acc()
            one = jnp.ones((16,), jnp.float32)

            def orow(r, _):
                for cc in range(D // 16):
                    rows0[r, pl.ds(cc * 16, 16)] = one
                return 0

            lax.fori_loop(0, k, orow, 0)
            plsc.subcore_barrier()

            def body2(i, _):
                base = s * PW + i * k
                pltpu.sync_copy(didx.at[pl.ds(base, k)], db0)
                remap(0, dest_only=True)
                pltpu.sync_copy(rows0, acc.at[db0], add=True)
                return 0

            lax.fori_loop(0, iters, body2, 0)
            if tail:
                load_tail(0, dest_only=True)
                remap(0, dest_only=True)
                pltpu.sync_copy(rows0, acc.at[db0], add=True)
            plsc.subcore_barrier()
            stage_out(cnt_o)

    return _p


_p1 = _make_pass(EACC, EHALF, None, True, True, 400)
_p2 = _make_pass(VACC, VHALF, (EHALF, EGAP), True, False, 400)
_p3 = _make_pass(EACC, EHALF, None, False, False, 400)
_p4 = _make_pass(VACC, VHALF, (EHALF, EGAP), False, False, 400)


def _mm_relu_body(x_ref, w_ref, b_ref, o_ref):
    o_ref[...] = jax.nn.relu(
        jnp.dot(x_ref[...], w_ref[...], preferred_element_type=jnp.float32)
        + b_ref[...])


def _relu_mm(x, w, b):
    n = x.shape[0]
    bs = 1000
    return pl.pallas_call(
        _mm_relu_body,
        grid=(n // bs,),
        in_specs=[pl.BlockSpec((bs, D), lambda i: (i, 0)),
                  pl.BlockSpec((D, D), lambda i: (0, 0)),
                  pl.BlockSpec((1, D), lambda i: (0, 0))],
        out_specs=pl.BlockSpec((bs, D), lambda i: (i, 0)),
        out_shape=jax.ShapeDtypeStruct((n, D), jnp.float32),
    )(x, w, b.reshape(1, D))


def _cnt_col(c_ref):
    return jnp.maximum(c_ref[0, :, :1], 1.0)


def _div_e_body(s_ref, c_ref, o_ref):
    o_ref[...] = s_ref[0] / _cnt_col(c_ref)


def _div_e(esum, ecnt):
    return pl.pallas_call(
        _div_e_body,
        grid=(NC,),
        in_specs=[pl.BlockSpec((1, EACC, D), lambda i: (i, 0, 0)),
                  pl.BlockSpec((1, EACC, D), lambda i: (i, 0, 0))],
        out_specs=pl.BlockSpec((EACC, D), lambda i: (i, 0)),
        out_shape=jax.ShapeDtypeStruct((NC * EACC, D), jnp.float32),
    )(esum, ecnt)


def _xw_body(s_ref, c_ref, w_ref, b_ref, o_ref):
    x = jax.nn.relu(s_ref[0] / _cnt_col(c_ref))[:VHALF]
    o_ref[...] = jax.nn.relu(
        jnp.dot(x, w_ref[...], preferred_element_type=jnp.float32)
        + b_ref[...])


def _xw(vsum, vcnt, w, b):
    return pl.pallas_call(
        _xw_body,
        grid=(NC,),
        in_specs=[pl.BlockSpec((1, VACC, D), lambda i: (i, 0, 0)),
                  pl.BlockSpec((1, VACC, D), lambda i: (i, 0, 0)),
                  pl.BlockSpec((D, D), lambda i: (0, 0)),
                  pl.BlockSpec((1, D), lambda i: (0, 0))],
        out_specs=pl.BlockSpec((VHALF, D), lambda i: (i, 0)),
        out_shape=jax.ShapeDtypeStruct((N_NODES, D), jnp.float32),
    )(vsum, vcnt, w, b.reshape(1, D))


def _feat_body(s_ref, c_ref, m_ref, o_ref):
    o_ref[...] = (s_ref[0] / _cnt_col(c_ref))[:VHALF] + m_ref[...]


def _feat_comb(vsum, vcnt, mol):
    return pl.pallas_call(
        _feat_body,
        grid=(NC,),
        in_specs=[pl.BlockSpec((1, VACC, D), lambda i: (i, 0, 0)),
                  pl.BlockSpec((1, VACC, D), lambda i: (i, 0, 0)),
                  pl.BlockSpec((VHALF, D), lambda i: (i, 0))],
        out_specs=pl.BlockSpec((VHALF, D), lambda i: (i, 0)),
        out_shape=jax.ShapeDtypeStruct((N_NODES, D), jnp.float32),
    )(vsum, vcnt, mol)


def _tail_body(f_ref, yb_ref, wmua, wmub, bmu, wlva, wlvb, blv,
               w3, b3, g2, beta, wc, bc, wda, wdb, bd,
               mu_ref, lv_ref, lg_ref, rc_ref):
    f = f_ref[...]
    yb = yb_ref[...]
    dot = functools.partial(jnp.dot, preferred_element_type=jnp.float32)
    mu = dot(f, wmua[...]) + dot(yb, wmub[...]) + bmu[...]
    lv = dot(f, wlva[...]) + dot(yb, wlvb[...]) + blv[...]
    h = dot(mu, w3[...]) + b3[...]
    h = jax.nn.relu(g2[...] * h + beta[...])
    lg = dot(h, wc[...]) + bc[...]
    rc = dot(mu, wda[...]) + dot(yb, wdb[...]) + bd[...]
    mu_ref[...] = mu
    lv_ref[...] = lv
    lg_ref[...] = lg
    rc_ref[...] = rc


def _tail(feat, y_bin, Wmu, bmu, Wlv, blv, W3, b3, gamma, beta, Wc, bc,
          Wd, bd):
    bs = 1000
    n = N_NODES - TRAIN
    g2 = (gamma / jnp.sqrt(1.0 + 1e-5)).reshape(1, 64)
    full = lambda *s: pl.BlockSpec(s, lambda i: tuple(0 for _ in s))
    return pl.pallas_call(
        _tail_body,
        grid=(n // bs,),
        in_specs=[
            pl.BlockSpec((bs, D), lambda i: (i + TRAIN // bs, 0)),
            pl.BlockSpec((bs, 3), lambda i: (i, 0)),
            full(D, 64), full(3, 64), full(1, 64),
            full(D, 64), full(3, 64), full(1, 64),
            full(64, 64), full(1, 64),
            full(1, 64), full(1, 64),
            full(64, 3), full(1, 3),
            full(64, D), full(3, D), full(1, D),
        ],
        out_specs=[pl.BlockSpec((bs, 64), lambda i: (i, 0)),
                   pl.BlockSpec((bs, 64), lambda i: (i, 0)),
                   pl.BlockSpec((bs, 3), lambda i: (i, 0)),
                   pl.BlockSpec((bs, D), lambda i: (i, 0))],
        out_shape=[jax.ShapeDtypeStruct((n, 64), jnp.float32),
                   jax.ShapeDtypeStruct((n, 64), jnp.float32),
                   jax.ShapeDtypeStruct((n, 3), jnp.float32),
                   jax.ShapeDtypeStruct((n, D), jnp.float32)],
    )(feat, y_bin,
      Wmu[:D], Wmu[D:], bmu.reshape(1, 64),
      Wlv[:D], Wlv[D:], blv.reshape(1, 64),
      W3, b3.reshape(1, 64), g2, beta.reshape(1, 64),
      Wc, bc.reshape(1, 3), Wd[:64], Wd[64:], bd.reshape(1, D))


def kernel(feature, v_idx, e_idx, y_bin, y_target, drug_matrix, new_data_idx,
           W1, b1, W2, b2, Wmu, bmu, Wlv, blv, W3, b3, gamma, beta, Wc, bc,
           Wd, bd):
    vi = v_idx.astype(jnp.int32)
    ei = e_idx.astype(jnp.int32)
    molidx = jnp.concatenate(
        [new_data_idx.astype(jnp.int32),
         jnp.zeros((MOLPAD - N_NODES,), jnp.int32)])

    A1 = _relu_mm(feature, W1, b1)
    es1, ec, mol = _p1(A1, vi, ei, drug_matrix, molidx)
    E1 = _div_e(es1, ec)
    vs1, vc = _p2(E1, ei, vi)
    A2 = _xw(vs1, vc, W2, b2)
    es2, = _p3(A2, vi, ei)
    E2 = _div_e(es2, ec)
    vs2, = _p4(E2, ei, vi)
    feat = _feat_comb(vs2, vc, mol[:N_NODES])
    mu, lv, lg, rc = _tail(feat, y_bin, Wmu, bmu, Wlv, blv, W3, b3,
                           gamma, beta, Wc, bc, Wd, bd)
    return (mu, lv, mu, lg, rc, y_target, feat)
